# baseline (speedup 1.0000x reference)
import functools

import jax
import jax.numpy as jnp
from jax import lax
from jax.experimental import pallas as pl
from jax.experimental.pallas import tpu as pltpu
from jax.experimental.pallas import tpu_sc as plsc

N = 10000
E = 320000
D_IN = 128
D_HID = 64
D_OUT = 128

NC = 2
NS = 16
NW = NC * NS
CHUNK = 128
CPW = 80
EPW = CHUNK * CPW
E_PAD = NW * EPW
NPAD = 10240
RPT = NPAD // NS

ROW_BLK = 400


def _make_sc_agg(with_count):
  mesh = plsc.VectorSubcoreMesh(core_axis_name="c", subcore_axis_name="s")
  out_type = [jax.ShapeDtypeStruct((NC, NPAD, 128), jnp.float32)]
  scratch = [
      pltpu.VMEM((2, CHUNK), jnp.int32),
      pltpu.VMEM((2, CHUNK), jnp.int32),
      pltpu.VMEM((CHUNK, 128), jnp.float32),
      pltpu.VMEM((CHUNK, 128), jnp.float32),
      pltpu.VMEM((16, 128), jnp.float32),
      pltpu.VMEM_SHARED((NPAD, 128), jnp.float32),
      pltpu.SemaphoreType.DMA,
      pltpu.SemaphoreType.DMA,
      pltpu.SemaphoreType.DMA,
      pltpu.SemaphoreType.DMA,
  ]
  if with_count:
    out_type.append(jax.ShapeDtypeStruct((NW, NPAD), jnp.float32))
    scratch += [
        pltpu.VMEM((NPAD,), jnp.float32),
    ]

  def body(p_hbm, edge_hbm, *rest):
    if with_count:
      (agg_out, cnt_out, idx0, idx1, rows0, rows1, zrow, sh_agg,
       sg0, sg1, si0, si1, cnt_loc) = rest
    else:
      (agg_out, idx0, idx1, rows0, rows1, zrow, sh_agg,
       sg0, sg1, si0, si1) = rest
    idx = (idx0, idx1)
    rows = (rows0, rows1)
    sg = (sg0, sg1)
    si = (si0, si1)

    core = lax.axis_index("c")
    sub = lax.axis_index("s")
    w = sub * NC + core

    zero16 = jnp.zeros((16,), jnp.float32)

    def zrow_body(i, carry):
      for j in range(8):
        zrow[i, 16 * j:16 * (j + 1)] = zero16
      return carry
    lax.fori_loop(0, 16, zrow_body, 0)

    def zcopy_body(k, carry):
      pltpu.sync_copy(zrow, sh_agg.at[pl.ds(sub * RPT + k * 16, 16)])
      return carry
    lax.fori_loop(0, RPT // 16, zcopy_body, 0)
    if with_count:
      def zcnt_body(i, carry):
        cnt_loc[pl.ds(i * 16, 16)] = zero16
        return carry
      lax.fori_loop(0, NPAD // 16, zcnt_body, 0)
    plsc.subcore_barrier()

    ones16 = jnp.ones((16,), jnp.float32)

    def idx_copy(c, b):
      return pltpu.make_async_copy(
          edge_hbm.at[:, pl.ds(w * EPW + c * CHUNK, CHUNK)], idx[b], si[b])

    def gather_copy(b):
      return pltpu.make_async_copy(p_hbm.at[idx[b].at[0]], rows[b], sg[b])

    pltpu.sync_copy(edge_hbm.at[:, pl.ds(w * EPW, CHUNK)], idx[0])
    gather_copy(0).start()
    idx_copy(1, 1).start()

    def pipe_body(t, carry):
      for b in range(2):
        c = 2 * t + b
        nb = 1 - b

        @pl.when(c + 1 < CPW)
        def _():
          idx_copy(c + 1, nb).wait()
          gather_copy(nb).start()

        gather_copy(b).wait()
        pltpu.sync_copy(rows[b], sh_agg.at[idx[b].at[1]], add=True)
        if with_count:
          for j in range(CHUNK // 16):
            dv = idx[b][1, pl.ds(16 * j, 16)]
            plsc.addupdate_scatter(cnt_loc, [dv], ones16)

        @pl.when(c + 2 < CPW)
        def _():
          idx_copy(c + 2, b).start()
      return carry
    lax.fori_loop(0, CPW // 2, pipe_body, 0)

    if with_count:
      pltpu.sync_copy(cnt_loc, cnt_out.at[w])
    plsc.subcore_barrier()
    pltpu.sync_copy(sh_agg.at[pl.ds(sub * RPT, RPT)],
                    agg_out.at[core, pl.ds(sub * RPT, RPT)])

  out_ty = tuple(out_type) if with_count else out_type[0]
  return pl.kernel(body, out_type=out_ty, mesh=mesh,
                   scratch_types=tuple(scratch),
                   compiler_params=pltpu.CompilerParams(
                       needs_layout_passes=False))


_sc_agg_cnt = _make_sc_agg(with_count=True)
_sc_agg = _make_sc_agg(with_count=False)


def _combine1_body(aggx_ref, cnt_ref, x_ref, wl1_ref, wr1_ref, b1_ref,
                   wl2_ref, h_ref, q_ref):
  a = aggx_ref[0] + aggx_ref[1]
  c = jnp.maximum(jnp.sum(cnt_ref[...], axis=0), 1.0)
  m = jnp.dot(a, wl1_ref[...], preferred_element_type=jnp.float32) / c
  h = jnp.maximum(
      m + b1_ref[...]
      + jnp.dot(x_ref[...], wr1_ref[...], preferred_element_type=jnp.float32),
      0.0)
  h_ref[...] = h
  q_ref[...] = jnp.dot(h, wl2_ref[...], preferred_element_type=jnp.float32)


def _combine1(aggx, cnt3, x, w_l1, w_r1, b1, w_l2):
  return pl.pallas_call(
      _combine1_body,
      grid=(N // ROW_BLK,),
      in_specs=[
          pl.BlockSpec((NC, ROW_BLK, D_IN), lambda i: (0, i, 0)),
          pl.BlockSpec((NW, ROW_BLK, 1), lambda i: (0, i, 0)),
          pl.BlockSpec((ROW_BLK, D_IN), lambda i: (i, 0)),
          pl.BlockSpec((D_IN, D_HID), lambda i: (0, 0)),
          pl.BlockSpec((D_IN, D_HID), lambda i: (0, 0)),
          pl.BlockSpec((1, D_HID), lambda i: (0, 0)),
          pl.BlockSpec((D_HID, D_OUT), lambda i: (0, 0)),
      ],
      out_specs=[
          pl.BlockSpec((ROW_BLK, D_HID), lambda i: (i, 0)),
          pl.BlockSpec((ROW_BLK, D_OUT), lambda i: (i, 0)),
      ],
      out_shape=[
          jax.ShapeDtypeStruct((N, D_HID), jnp.float32),
          jax.ShapeDtypeStruct((N, D_OUT), jnp.float32),
      ],
  )(aggx, cnt3, x, w_l1, w_r1, b1, w_l2)


def _combine2_body(aggq_ref, cnt_ref, h_ref, wr2_ref, b2_ref, out_ref):
  a = aggq_ref[0] + aggq_ref[1]
  c = jnp.maximum(jnp.sum(cnt_ref[...], axis=0), 1.0)
  out_ref[...] = (
      a / c + b2_ref[...]
      + jnp.dot(h_ref[...], wr2_ref[...], preferred_element_type=jnp.float32))


def _combine2(aggq, cnt3, h, w_r2, b2):
  return pl.pallas_call(
      _combine2_body,
      grid=(N // ROW_BLK,),
      in_specs=[
          pl.BlockSpec((NC, ROW_BLK, D_OUT), lambda i: (0, i, 0)),
          pl.BlockSpec((NW, ROW_BLK, 1), lambda i: (0, i, 0)),
          pl.BlockSpec((ROW_BLK, D_HID), lambda i: (i, 0)),
          pl.BlockSpec((D_HID, D_OUT), lambda i: (0, 0)),
          pl.BlockSpec((1, D_OUT), lambda i: (0, 0)),
      ],
      out_specs=pl.BlockSpec((ROW_BLK, D_OUT), lambda i: (i, 0)),
      out_shape=jax.ShapeDtypeStruct((N, D_OUT), jnp.float32),
  )(aggq, cnt3, h, w_r2, b2)


def kernel(x, edge_index, W_l1, b_l1, W_r1, W_l2, b_l2, W_r2):
  pad = E_PAD - E
  pad_block = jnp.stack([jnp.zeros((pad,), jnp.int32),
                         jnp.full((pad,), NPAD - 1, jnp.int32)])
  edge2 = jnp.concatenate([edge_index.astype(jnp.int32), pad_block], axis=1)

  aggx, cntp = _sc_agg_cnt(x, edge2)
  cnt3 = cntp.reshape(NW, NPAD, 1)
  h, q = _combine1(aggx, cnt3, x, W_l1, W_r1, b_l1.reshape(1, D_HID), W_l2)

  aggq = _sc_agg(q, edge2)
  out = _combine2(aggq, cnt3, h, W_r2, b_l2.reshape(1, D_OUT))
  return out


# speedup vs baseline: 1.0199x; 1.0199x over previous
import functools

import jax
import jax.numpy as jnp
from jax import lax
from jax.experimental import pallas as pl
from jax.experimental.pallas import tpu as pltpu
from jax.experimental.pallas import tpu_sc as plsc

N = 10000
E = 320000
D_IN = 128
D_HID = 64
D_OUT = 128

NC = 2
NS = 16
NW = NC * NS
CHUNK = 128
CPW0 = 124
CPW1 = 36
CPT = CPW0 + CPW1
E_PAD = NS * CPT * CHUNK
NPAD = 10240
RPT = NPAD // NS

ROW_BLK = 400


def _make_sc_agg(with_count):
  mesh = plsc.VectorSubcoreMesh(core_axis_name="c", subcore_axis_name="s")
  out_type = [jax.ShapeDtypeStruct((NC, NPAD, 128), jnp.float32)]
  scratch = [
      pltpu.VMEM((2, CHUNK), jnp.int32),
      pltpu.VMEM((2, CHUNK), jnp.int32),
      pltpu.VMEM((CHUNK, 128), jnp.float32),
      pltpu.VMEM((CHUNK, 128), jnp.float32),
      pltpu.VMEM((16, 128), jnp.float32),
      pltpu.VMEM_SHARED((NPAD, 128), jnp.float32),
      pltpu.SemaphoreType.DMA,
      pltpu.SemaphoreType.DMA,
      pltpu.SemaphoreType.DMA,
      pltpu.SemaphoreType.DMA,
  ]
  if with_count:
    out_type.append(jax.ShapeDtypeStruct((NW, NPAD), jnp.float32))
    scratch += [
        pltpu.VMEM((NPAD,), jnp.float32),
    ]

  def body(p_hbm, edge_hbm, *rest):
    if with_count:
      (agg_out, cnt_out, idx0, idx1, rows0, rows1, zrow, sh_agg,
       sg0, sg1, si0, si1, cnt_loc) = rest
    else:
      (agg_out, idx0, idx1, rows0, rows1, zrow, sh_agg,
       sg0, sg1, si0, si1) = rest
    idx = (idx0, idx1)
    rows = (rows0, rows1)
    sg = (sg0, sg1)
    si = (si0, si1)

    core = lax.axis_index("c")
    sub = lax.axis_index("s")
    w = sub * NC + core
    cpw = jnp.where(core == 0, CPW0, CPW1)
    base_chunk = jnp.where(core == 0, sub * CPW0, NS * CPW0 + sub * CPW1)

    zero16 = jnp.zeros((16,), jnp.float32)

    def zrow_body(i, carry):
      for j in range(8):
        zrow[i, 16 * j:16 * (j + 1)] = zero16
      return carry
    lax.fori_loop(0, 16, zrow_body, 0)

    def zcopy_body(k, carry):
      pltpu.sync_copy(zrow, sh_agg.at[pl.ds(sub * RPT + k * 16, 16)])
      return carry
    lax.fori_loop(0, RPT // 16, zcopy_body, 0)
    if with_count:
      def zcnt_body(i, carry):
        cnt_loc[pl.ds(i * 16, 16)] = zero16
        return carry
      lax.fori_loop(0, NPAD // 16, zcnt_body, 0)
    plsc.subcore_barrier()

    ones16 = jnp.ones((16,), jnp.float32)

    def idx_copy(c, b):
      return pltpu.make_async_copy(
          edge_hbm.at[:, pl.ds((base_chunk + c) * CHUNK, CHUNK)], idx[b], si[b])

    def gather_copy(b):
      return pltpu.make_async_copy(p_hbm.at[idx[b].at[0]], rows[b], sg[b])

    pltpu.sync_copy(edge_hbm.at[:, pl.ds(base_chunk * CHUNK, CHUNK)], idx[0])
    gather_copy(0).start()
    idx_copy(1, 1).start()

    def pipe_body(t, carry):
      for b in range(2):
        c = 2 * t + b
        nb = 1 - b

        @pl.when(c + 1 < cpw)
        def _():
          idx_copy(c + 1, nb).wait()
          gather_copy(nb).start()

        gather_copy(b).wait()
        pltpu.sync_copy(rows[b], sh_agg.at[idx[b].at[1]], add=True)
        if with_count:
          for j in range(CHUNK // 16):
            dv = idx[b][1, pl.ds(16 * j, 16)]
            plsc.addupdate_scatter(cnt_loc, [dv], ones16)

        @pl.when(c + 2 < cpw)
        def _():
          idx_copy(c + 2, b).start()
      return carry
    lax.fori_loop(0, cpw // 2, pipe_body, 0)

    if with_count:
      pltpu.sync_copy(cnt_loc, cnt_out.at[w])
    plsc.subcore_barrier()
    pltpu.sync_copy(sh_agg.at[pl.ds(sub * RPT, RPT)],
                    agg_out.at[core, pl.ds(sub * RPT, RPT)])

  out_ty = tuple(out_type) if with_count else out_type[0]
  return pl.kernel(body, out_type=out_ty, mesh=mesh,
                   scratch_types=tuple(scratch),
                   compiler_params=pltpu.CompilerParams(
                       needs_layout_passes=False))


_sc_agg_cnt = _make_sc_agg(with_count=True)
_sc_agg = _make_sc_agg(with_count=False)


def _combine1_body(aggx_ref, cnt_ref, x_ref, wl1_ref, wr1_ref, b1_ref,
                   wl2_ref, h_ref, q_ref):
  a = aggx_ref[0] + aggx_ref[1]
  c = jnp.maximum(jnp.sum(cnt_ref[...], axis=0), 1.0)
  m = jnp.dot(a, wl1_ref[...], preferred_element_type=jnp.float32) / c
  h = jnp.maximum(
      m + b1_ref[...]
      + jnp.dot(x_ref[...], wr1_ref[...], preferred_element_type=jnp.float32),
      0.0)
  h_ref[...] = h
  q_ref[...] = jnp.dot(h, wl2_ref[...], preferred_element_type=jnp.float32)


def _combine1(aggx, cnt3, x, w_l1, w_r1, b1, w_l2):
  return pl.pallas_call(
      _combine1_body,
      grid=(N // ROW_BLK,),
      in_specs=[
          pl.BlockSpec((NC, ROW_BLK, D_IN), lambda i: (0, i, 0)),
          pl.BlockSpec((NW, ROW_BLK, 1), lambda i: (0, i, 0)),
          pl.BlockSpec((ROW_BLK, D_IN), lambda i: (i, 0)),
          pl.BlockSpec((D_IN, D_HID), lambda i: (0, 0)),
          pl.BlockSpec((D_IN, D_HID), lambda i: (0, 0)),
          pl.BlockSpec((1, D_HID), lambda i: (0, 0)),
          pl.BlockSpec((D_HID, D_OUT), lambda i: (0, 0)),
      ],
      out_specs=[
          pl.BlockSpec((ROW_BLK, D_HID), lambda i: (i, 0)),
          pl.BlockSpec((ROW_BLK, D_OUT), lambda i: (i, 0)),
      ],
      out_shape=[
          jax.ShapeDtypeStruct((N, D_HID), jnp.float32),
          jax.ShapeDtypeStruct((N, D_OUT), jnp.float32),
      ],
  )(aggx, cnt3, x, w_l1, w_r1, b1, w_l2)


def _combine2_body(aggq_ref, cnt_ref, h_ref, wr2_ref, b2_ref, out_ref):
  a = aggq_ref[0] + aggq_ref[1]
  c = jnp.maximum(jnp.sum(cnt_ref[...], axis=0), 1.0)
  out_ref[...] = (
      a / c + b2_ref[...]
      + jnp.dot(h_ref[...], wr2_ref[...], preferred_element_type=jnp.float32))


def _combine2(aggq, cnt3, h, w_r2, b2):
  return pl.pallas_call(
      _combine2_body,
      grid=(N // ROW_BLK,),
      in_specs=[
          pl.BlockSpec((NC, ROW_BLK, D_OUT), lambda i: (0, i, 0)),
          pl.BlockSpec((NW, ROW_BLK, 1), lambda i: (0, i, 0)),
          pl.BlockSpec((ROW_BLK, D_HID), lambda i: (i, 0)),
          pl.BlockSpec((D_HID, D_OUT), lambda i: (0, 0)),
          pl.BlockSpec((1, D_OUT), lambda i: (0, 0)),
      ],
      out_specs=pl.BlockSpec((ROW_BLK, D_OUT), lambda i: (i, 0)),
      out_shape=jax.ShapeDtypeStruct((N, D_OUT), jnp.float32),
  )(aggq, cnt3, h, w_r2, b2)


def kernel(x, edge_index, W_l1, b_l1, W_r1, W_l2, b_l2, W_r2):
  pad = E_PAD - E
  pad_block = jnp.stack([jnp.zeros((pad,), jnp.int32),
                         jnp.full((pad,), NPAD - 1, jnp.int32)])
  edge2 = jnp.concatenate([edge_index.astype(jnp.int32), pad_block], axis=1)

  aggx, cntp = _sc_agg_cnt(x, edge2)
  cnt3 = cntp.reshape(NW, NPAD, 1)
  h, q = _combine1(aggx, cnt3, x, W_l1, W_r1, b_l1.reshape(1, D_HID), W_l2)

  aggq = _sc_agg(q, edge2)
  out = _combine2(aggq, cnt3, h, W_r2, b_l2.reshape(1, D_OUT))
  return out
